# baseline (speedup 1.0000x reference)
import jax
import jax.numpy as jnp
from jax.experimental import pallas as pl

_N = 3276800
_LANES = 128
_ROWS = (_N * 2) // _LANES
_BLOCK_R = 2048


def _ew_kernel(x_ref, c1_ref, c2e_ref, c2o_ref, c3_ref, o_ref):
    x = x_ref[...].reshape(_BLOCK_R // 8, 8, _LANES)
    c1 = c1_ref[...].reshape(1, 8, _LANES)
    c2e = c2e_ref[...].reshape(1, 8, _LANES)
    c2o = c2o_ref[...].reshape(1, 8, _LANES)
    c3 = c3_ref[...].reshape(1, 8, _LANES)
    xd = jnp.roll(x, -1, axis=1)
    xu = jnp.roll(x, 1, axis=1)
    out = x * c1 + xd * c2e + xu * c2o + c3
    o_ref[...] = out.reshape(_BLOCK_R, _LANES)


def kernel(keys, W, b):
    x = keys.reshape(25600, 128, 2).transpose(0, 2, 1).reshape(_ROWS, _LANES)
    even = (jnp.arange(8) % 2 == 0)[:, None]
    zeros = jnp.zeros((8, 1), jnp.float32)
    c1 = jnp.broadcast_to(jnp.where(even, W[0, 0], W[1, 1]), (8, _LANES))
    c2e = jnp.broadcast_to(jnp.where(even, W[0, 1], zeros), (8, _LANES))
    c2o = jnp.broadcast_to(jnp.where(even, zeros, W[1, 0]), (8, _LANES))
    c3 = jnp.broadcast_to(jnp.where(even, b[0], b[1]), (8, _LANES))
    cspec = pl.BlockSpec((8, _LANES), lambda i: (0, 0))
    out = pl.pallas_call(
        _ew_kernel,
        grid=(_ROWS // _BLOCK_R,),
        in_specs=[
            pl.BlockSpec((_BLOCK_R, _LANES), lambda i: (i, 0)),
            cspec, cspec, cspec, cspec,
        ],
        out_specs=pl.BlockSpec((_BLOCK_R, _LANES), lambda i: (i, 0)),
        out_shape=jax.ShapeDtypeStruct((_ROWS, _LANES), jnp.float32),
    )(x, c1, c2e, c2o, c3)
    return out.reshape(25600, 2, 128).transpose(0, 2, 1).reshape(_N, 2)


# speedup vs baseline: 1.3045x; 1.3045x over previous
import jax
import jax.numpy as jnp
from jax.experimental import pallas as pl

_N = 3276800
_LANES = 128
_ROWS = (_N * 2) // _LANES
_BLOCK_R = 6400


def _ew_kernel(x_ref, c1_ref, c2e_ref, c2o_ref, c3_ref, o_ref):
    x = x_ref[...].reshape(_BLOCK_R // 8, 8, _LANES)
    c1 = c1_ref[...].reshape(1, 8, _LANES)
    c2e = c2e_ref[...].reshape(1, 8, _LANES)
    c2o = c2o_ref[...].reshape(1, 8, _LANES)
    c3 = c3_ref[...].reshape(1, 8, _LANES)
    xd = jnp.roll(x, -1, axis=1)
    xu = jnp.roll(x, 1, axis=1)
    out = x * c1 + xd * c2e + xu * c2o + c3
    o_ref[...] = out.reshape(_BLOCK_R, _LANES)


def kernel(keys, W, b):
    x = keys.reshape(25600, 128, 2).transpose(0, 2, 1).reshape(_ROWS, _LANES)
    even = (jnp.arange(8) % 2 == 0)[:, None]
    zeros = jnp.zeros((8, 1), jnp.float32)
    c1 = jnp.broadcast_to(jnp.where(even, W[0, 0], W[1, 1]), (8, _LANES))
    c2e = jnp.broadcast_to(jnp.where(even, W[0, 1], zeros), (8, _LANES))
    c2o = jnp.broadcast_to(jnp.where(even, zeros, W[1, 0]), (8, _LANES))
    c3 = jnp.broadcast_to(jnp.where(even, b[0], b[1]), (8, _LANES))
    cspec = pl.BlockSpec((8, _LANES), lambda i: (0, 0))
    out = pl.pallas_call(
        _ew_kernel,
        grid=(_ROWS // _BLOCK_R,),
        in_specs=[
            pl.BlockSpec((_BLOCK_R, _LANES), lambda i: (i, 0)),
            cspec, cspec, cspec, cspec,
        ],
        out_specs=pl.BlockSpec((_BLOCK_R, _LANES), lambda i: (i, 0)),
        out_shape=jax.ShapeDtypeStruct((_ROWS, _LANES), jnp.float32),
    )(x, c1, c2e, c2o, c3)
    return out.reshape(25600, 2, 128).transpose(0, 2, 1).reshape(_N, 2)


# speedup vs baseline: 1.3445x; 1.0307x over previous
import jax
import jax.numpy as jnp
from jax.experimental import pallas as pl

_N = 3276800
_LANES = 128
_ROWS = (_N * 2) // _LANES
_BLOCK_R = 12800


def _ew_kernel(x_ref, c1_ref, c2e_ref, c2o_ref, c3_ref, o_ref):
    x = x_ref[...].reshape(_BLOCK_R // 8, 8, _LANES)
    c1 = c1_ref[...].reshape(1, 8, _LANES)
    c2e = c2e_ref[...].reshape(1, 8, _LANES)
    c2o = c2o_ref[...].reshape(1, 8, _LANES)
    c3 = c3_ref[...].reshape(1, 8, _LANES)
    xd = jnp.roll(x, -1, axis=1)
    xu = jnp.roll(x, 1, axis=1)
    out = x * c1 + xd * c2e + xu * c2o + c3
    o_ref[...] = out.reshape(_BLOCK_R, _LANES)


def kernel(keys, W, b):
    x = keys.reshape(25600, 128, 2).transpose(0, 2, 1).reshape(_ROWS, _LANES)
    even = (jnp.arange(8) % 2 == 0)[:, None]
    zeros = jnp.zeros((8, 1), jnp.float32)
    c1 = jnp.broadcast_to(jnp.where(even, W[0, 0], W[1, 1]), (8, _LANES))
    c2e = jnp.broadcast_to(jnp.where(even, W[0, 1], zeros), (8, _LANES))
    c2o = jnp.broadcast_to(jnp.where(even, zeros, W[1, 0]), (8, _LANES))
    c3 = jnp.broadcast_to(jnp.where(even, b[0], b[1]), (8, _LANES))
    cspec = pl.BlockSpec((8, _LANES), lambda i: (0, 0))
    out = pl.pallas_call(
        _ew_kernel,
        grid=(_ROWS // _BLOCK_R,),
        in_specs=[
            pl.BlockSpec((_BLOCK_R, _LANES), lambda i: (i, 0)),
            cspec, cspec, cspec, cspec,
        ],
        out_specs=pl.BlockSpec((_BLOCK_R, _LANES), lambda i: (i, 0)),
        out_shape=jax.ShapeDtypeStruct((_ROWS, _LANES), jnp.float32),
    )(x, c1, c2e, c2o, c3)
    return out.reshape(25600, 2, 128).transpose(0, 2, 1).reshape(_N, 2)


# speedup vs baseline: 1.4370x; 1.0688x over previous
import jax
import jax.numpy as jnp
from jax.experimental import pallas as pl

_N = 3276800
_LANES = 128
_ROWS = (_N * 2) // _LANES
_BLOCK_R = 25600


def _ew_kernel(x_ref, c1_ref, c2e_ref, c2o_ref, c3_ref, o_ref):
    x = x_ref[...].reshape(_BLOCK_R // 8, 8, _LANES)
    c1 = c1_ref[...].reshape(1, 8, _LANES)
    c2e = c2e_ref[...].reshape(1, 8, _LANES)
    c2o = c2o_ref[...].reshape(1, 8, _LANES)
    c3 = c3_ref[...].reshape(1, 8, _LANES)
    xd = jnp.roll(x, -1, axis=1)
    xu = jnp.roll(x, 1, axis=1)
    out = x * c1 + xd * c2e + xu * c2o + c3
    o_ref[...] = out.reshape(_BLOCK_R, _LANES)


def kernel(keys, W, b):
    x = keys.reshape(25600, 128, 2).transpose(0, 2, 1).reshape(_ROWS, _LANES)
    even = (jnp.arange(8) % 2 == 0)[:, None]
    zeros = jnp.zeros((8, 1), jnp.float32)
    c1 = jnp.broadcast_to(jnp.where(even, W[0, 0], W[1, 1]), (8, _LANES))
    c2e = jnp.broadcast_to(jnp.where(even, W[0, 1], zeros), (8, _LANES))
    c2o = jnp.broadcast_to(jnp.where(even, zeros, W[1, 0]), (8, _LANES))
    c3 = jnp.broadcast_to(jnp.where(even, b[0], b[1]), (8, _LANES))
    cspec = pl.BlockSpec((8, _LANES), lambda i: (0, 0))
    out = pl.pallas_call(
        _ew_kernel,
        grid=(_ROWS // _BLOCK_R,),
        in_specs=[
            pl.BlockSpec((_BLOCK_R, _LANES), lambda i: (i, 0)),
            cspec, cspec, cspec, cspec,
        ],
        out_specs=pl.BlockSpec((_BLOCK_R, _LANES), lambda i: (i, 0)),
        out_shape=jax.ShapeDtypeStruct((_ROWS, _LANES), jnp.float32),
    )(x, c1, c2e, c2o, c3)
    return out.reshape(25600, 2, 128).transpose(0, 2, 1).reshape(_N, 2)
